# speedup vs baseline: 5.7237x; 5.7237x over previous
import numpy as np
import jax
import jax.numpy as jnp
from jax.experimental import pallas as pl
from jax.experimental.pallas import tpu as pltpu

_MAXD = 7
_H = 128
_PF = 6


def _deint(x):
    x = x & 0x55555555
    x = (x | (x >> 1)) & 0x33333333
    x = (x | (x >> 2)) & 0x0F0F0F0F
    x = (x | (x >> 4)) & 0x00FF00FF
    x = (x | (x >> 8)) & 0x0000FFFF
    return x


def _pos_const(d):
    keys = np.arange(4 ** d, dtype=np.int64)
    ix = _deint(keys).astype(np.float32)
    iy = _deint(keys >> 1).astype(np.float32)
    res = np.float32(1 << d)
    x = (ix + np.float32(0.5)) / res
    y = (iy + np.float32(0.5)) / res
    dn = np.full_like(x, np.float32(d) / np.float32(_MAXD))
    pos = np.stack([x, y, dn], axis=1).astype(np.float32)
    freqs = (np.float32(2.0) ** np.arange(_PF, dtype=np.float32)).reshape(1, 1, -1)
    xx = pos[..., None] * np.pi * 2.0 * freqs
    enc = np.concatenate([np.sin(xx), np.cos(xx)], axis=-1).astype(np.float32)
    enc = enc.reshape(pos.shape[0], -1)
    return np.concatenate([pos, enc], axis=1).astype(np.float32)


_POS = [_pos_const(d) for d in range(_MAXD + 1)]


def _shift(a, d, mode):
    c = a.shape[1]
    if d == 0:
        return jnp.zeros_like(a)
    b = a.reshape(4 ** (d - 1), 4 * c)
    q0 = b[:, 0 * c:1 * c]
    q1 = b[:, 1 * c:2 * c]
    q2 = b[:, 2 * c:3 * c]
    q3 = b[:, 3 * c:4 * c]
    if mode == "px":
        e = _shift(jnp.concatenate([q0, q2], axis=1), d - 1, mode)
        o0, o2 = q1, q3
        o1, o3 = e[:, :c], e[:, c:]
    elif mode == "mx":
        e = _shift(jnp.concatenate([q1, q3], axis=1), d - 1, mode)
        o1, o3 = q0, q2
        o0, o2 = e[:, :c], e[:, c:]
    elif mode == "py":
        e = _shift(jnp.concatenate([q0, q1], axis=1), d - 1, mode)
        o0, o1 = q2, q3
        o2, o3 = e[:, :c], e[:, c:]
    else:
        e = _shift(jnp.concatenate([q2, q3], axis=1), d - 1, mode)
        o2, o3 = q0, q1
        o0, o1 = e[:, :c], e[:, c:]
    return jnp.concatenate([o0, o1, o2, o3], axis=1).reshape(4 ** d, c)


def _mm(a, b):
    return jax.lax.dot_general(a, b, (((1,), (0,)), ((), ())),
                               preferred_element_type=jnp.float32)


def _conv3x3(h, d, cw_ref, cb_ref):
    xm = _shift(h, d, "mx")
    xp = _shift(h, d, "px")
    taps = (
        _shift(xm, d, "my"), _shift(h, d, "my"), _shift(xp, d, "my"),
        xm, h, xp,
        _shift(xm, d, "py"), _shift(h, d, "py"), _shift(xp, d, "py"),
    )
    acc = jnp.broadcast_to(cb_ref[d].reshape(1, _H), (h.shape[0], _H))
    for k, g in enumerate(taps):
        acc = acc + _mm(g, cw_ref[d, k * _H:(k + 1) * _H, :])
    return jnp.maximum(acc, 0.0)


def _emb_ln(h, d, ew_ref, eb_ref, lg_ref, lb_ref, dg_ref):
    z = _mm(h, ew_ref[d]) + eb_ref[d].reshape(1, _H)
    mu = jnp.mean(z, axis=-1, keepdims=True)
    zc = z - mu
    var = jnp.mean(zc * zc, axis=-1, keepdims=True)
    zn = zc * jax.lax.rsqrt(var + 1e-5)
    return (zn * lg_ref[d].reshape(1, _H) + lb_ref[d].reshape(1, _H)) * dg_ref[d]


def _body(x0, x1, x2, x3, x4, x5, x6, x7, wp_ref, bp_ref, cw_ref, cb_ref,
          ew_ref, eb_ref, lg_ref, lb_ref, dg_ref,
          o0, o1, o2, o3, o4, o5, o6, o7):
    xs = (x0, x1, x2, x3, x4, x5, x6, x7)
    outs = (o0, o1, o2, o3, o4, o5, o6, o7)
    wp = wp_ref[:]
    bp = bp_ref[:]
    pooled = None
    for d in range(_MAXD, -1, -1):
        h = _mm(xs[d][:], wp) + bp
        if pooled is not None:
            h = h + pooled
        if 1 <= d <= 6:
            h = _conv3x3(h, d, cw_ref, cb_ref)
        outs[d][:] = _emb_ln(h, d, ew_ref, eb_ref, lg_ref, lb_ref, dg_ref)
        if d > 0:
            hp = h.reshape(4 ** (d - 1), 4 * _H)
            pooled = (hp[:, 0 * _H:1 * _H] + hp[:, 1 * _H:2 * _H]
                      + hp[:, 2 * _H:3 * _H] + hp[:, 3 * _H:4 * _H]) * 0.25


def kernel(features_0, features_1, features_2, features_3, features_4,
           features_5, features_6, features_7, in_proj_W, in_proj_b,
           conv_W, conv_b, emb_W, emb_b, ln_g, ln_b, depth_gain):
    feats = (features_0, features_1, features_2, features_3, features_4,
             features_5, features_6, features_7)
    xs = [jnp.concatenate([f, jnp.asarray(_POS[d])], axis=1)
          for d, f in enumerate(feats)]
    vmem = pl.BlockSpec(memory_space=pltpu.MemorySpace.VMEM)
    smem = pl.BlockSpec(memory_space=pltpu.MemorySpace.SMEM)
    outs = pl.pallas_call(
        _body,
        out_shape=[jax.ShapeDtypeStruct((4 ** d, _H), jnp.float32)
                   for d in range(_MAXD + 1)],
        in_specs=[vmem] * 16 + [smem],
        out_specs=[vmem] * (_MAXD + 1),
    )(*xs, in_proj_W, in_proj_b.reshape(1, _H), conv_W, conv_b,
      emb_W, emb_b, ln_g, ln_b, depth_gain)
    return tuple(outs)


# speedup vs baseline: 5.9861x; 1.0459x over previous
import numpy as np
import jax
import jax.numpy as jnp
from jax.experimental import pallas as pl
from jax.experimental.pallas import tpu as pltpu

_MAXD = 7
_H = 128
_PF = 6


def _deint(x):
    x = x & 0x55555555
    x = (x | (x >> 1)) & 0x33333333
    x = (x | (x >> 2)) & 0x0F0F0F0F
    x = (x | (x >> 4)) & 0x00FF00FF
    x = (x | (x >> 8)) & 0x0000FFFF
    return x


def _pos_const(d):
    keys = np.arange(4 ** d, dtype=np.int64)
    ix = _deint(keys).astype(np.float32)
    iy = _deint(keys >> 1).astype(np.float32)
    res = np.float32(1 << d)
    x = (ix + np.float32(0.5)) / res
    y = (iy + np.float32(0.5)) / res
    dn = np.full_like(x, np.float32(d) / np.float32(_MAXD))
    pos = np.stack([x, y, dn], axis=1).astype(np.float32)
    freqs = (np.float32(2.0) ** np.arange(_PF, dtype=np.float32)).reshape(1, 1, -1)
    xx = pos[..., None] * np.pi * 2.0 * freqs
    enc = np.concatenate([np.sin(xx), np.cos(xx)], axis=-1).astype(np.float32)
    enc = enc.reshape(pos.shape[0], -1)
    return np.concatenate([pos, enc], axis=1).astype(np.float32)


_POS = [_pos_const(d) for d in range(_MAXD + 1)]


def _shift(a, d, mode):
    c = a.shape[1]
    if d == 0:
        return jnp.zeros_like(a)
    b = a.reshape(4 ** (d - 1), 4 * c)
    q0 = b[:, 0 * c:1 * c]
    q1 = b[:, 1 * c:2 * c]
    q2 = b[:, 2 * c:3 * c]
    q3 = b[:, 3 * c:4 * c]
    if mode == "px":
        e = _shift(jnp.concatenate([q0, q2], axis=1), d - 1, mode)
        o0, o2 = q1, q3
        o1, o3 = e[:, :c], e[:, c:]
    elif mode == "mx":
        e = _shift(jnp.concatenate([q1, q3], axis=1), d - 1, mode)
        o1, o3 = q0, q2
        o0, o2 = e[:, :c], e[:, c:]
    elif mode == "py":
        e = _shift(jnp.concatenate([q0, q1], axis=1), d - 1, mode)
        o0, o1 = q2, q3
        o2, o3 = e[:, :c], e[:, c:]
    else:
        e = _shift(jnp.concatenate([q2, q3], axis=1), d - 1, mode)
        o2, o3 = q0, q1
        o0, o1 = e[:, :c], e[:, c:]
    return jnp.concatenate([o0, o1, o2, o3], axis=1).reshape(4 ** d, c)


def _mm(a, b):
    return jax.lax.dot_general(a, b, (((1,), (0,)), ((), ())),
                               preferred_element_type=jnp.float32)


def _conv3x3(h, d, cw_ref, cb_ref):
    xm = _shift(h, d, "mx")
    xp = _shift(h, d, "px")
    rows = []
    for r in range(3):
        t = (_mm(xm, cw_ref[d - 1, (3 * r + 0) * _H:(3 * r + 1) * _H, :])
             + _mm(h, cw_ref[d - 1, (3 * r + 1) * _H:(3 * r + 2) * _H, :])
             + _mm(xp, cw_ref[d - 1, (3 * r + 2) * _H:(3 * r + 3) * _H, :]))
        rows.append(t)
    acc = (rows[1] + _shift(rows[0], d, "my") + _shift(rows[2], d, "py")
           + cb_ref[d].reshape(1, _H))
    return jnp.maximum(acc, 0.0)


def _emb_ln(h, d, ew_ref, eb_ref, lg_ref, lb_ref, dg_ref):
    z = _mm(h, ew_ref[d]) + eb_ref[d].reshape(1, _H)
    mu = jnp.mean(z, axis=-1, keepdims=True)
    zc = z - mu
    var = jnp.mean(zc * zc, axis=-1, keepdims=True)
    zn = zc * jax.lax.rsqrt(var + 1e-5)
    return (zn * lg_ref[d].reshape(1, _H) + lb_ref[d].reshape(1, _H)) * dg_ref[d]


def _body(x0, x1, x2, x3, x4, x5, x6, x7, wp_ref, bp_ref, cw_ref, cb_ref,
          ew_ref, eb_ref, lg_ref, lb_ref, dg_ref,
          o0, o1, o2, o3, o4, o5, o6, o7):
    xs = (x0, x1, x2, x3, x4, x5, x6, x7)
    outs = (o0, o1, o2, o3, o4, o5, o6, o7)
    wp = wp_ref[:]
    bp = bp_ref[:]
    pooled = None
    for d in range(_MAXD, -1, -1):
        h = _mm(xs[d][:], wp) + bp
        if pooled is not None:
            h = h + pooled
        if 1 <= d <= 6:
            h = _conv3x3(h, d, cw_ref, cb_ref)
        outs[d][:] = _emb_ln(h, d, ew_ref, eb_ref, lg_ref, lb_ref, dg_ref)
        if d > 0:
            hp = h.reshape(4 ** (d - 1), 4 * _H)
            pooled = (hp[:, 0 * _H:1 * _H] + hp[:, 1 * _H:2 * _H]
                      + hp[:, 2 * _H:3 * _H] + hp[:, 3 * _H:4 * _H]) * 0.25


def kernel(features_0, features_1, features_2, features_3, features_4,
           features_5, features_6, features_7, in_proj_W, in_proj_b,
           conv_W, conv_b, emb_W, emb_b, ln_g, ln_b, depth_gain):
    feats = (features_0, features_1, features_2, features_3, features_4,
             features_5, features_6, features_7)
    xs = [jnp.concatenate([f, jnp.asarray(_POS[d])], axis=1)
          for d, f in enumerate(feats)]
    vmem = pl.BlockSpec(memory_space=pltpu.MemorySpace.VMEM)
    smem = pl.BlockSpec(memory_space=pltpu.MemorySpace.SMEM)
    outs = pl.pallas_call(
        _body,
        out_shape=[jax.ShapeDtypeStruct((4 ** d, _H), jnp.float32)
                   for d in range(_MAXD + 1)],
        in_specs=[vmem] * 16 + [smem],
        out_specs=[vmem] * (_MAXD + 1),
    )(*xs, in_proj_W, in_proj_b.reshape(1, _H), conv_W[1:7], conv_b,
      emb_W, emb_b, ln_g, ln_b, depth_gain)
    return tuple(outs)


# speedup vs baseline: 6.2976x; 1.0520x over previous
import numpy as np
import jax
import jax.numpy as jnp
from jax.experimental import pallas as pl
from jax.experimental.pallas import tpu as pltpu

_MAXD = 7
_H = 128
_PF = 6


def _deint(x):
    x = x & 0x55555555
    x = (x | (x >> 1)) & 0x33333333
    x = (x | (x >> 2)) & 0x0F0F0F0F
    x = (x | (x >> 4)) & 0x00FF00FF
    x = (x | (x >> 8)) & 0x0000FFFF
    return x


def _pos_const(d):
    keys = np.arange(4 ** d, dtype=np.int64)
    ix = _deint(keys).astype(np.float32)
    iy = _deint(keys >> 1).astype(np.float32)
    res = np.float32(1 << d)
    x = (ix + np.float32(0.5)) / res
    y = (iy + np.float32(0.5)) / res
    dn = np.full_like(x, np.float32(d) / np.float32(_MAXD))
    pos = np.stack([x, y, dn], axis=1).astype(np.float32)
    freqs = (np.float32(2.0) ** np.arange(_PF, dtype=np.float32)).reshape(1, 1, -1)
    xx = pos[..., None] * np.pi * 2.0 * freqs
    enc = np.concatenate([np.sin(xx), np.cos(xx)], axis=-1).astype(np.float32)
    enc = enc.reshape(pos.shape[0], -1)
    return np.concatenate([pos, enc], axis=1).astype(np.float32)


_POS = [_pos_const(d) for d in range(_MAXD + 1)]


def _shift(a, d, mode):
    c = a.shape[1]
    if d == 0:
        return jnp.zeros_like(a)
    b = a.reshape(4 ** (d - 1), 4 * c)
    q0 = b[:, 0 * c:1 * c]
    q1 = b[:, 1 * c:2 * c]
    q2 = b[:, 2 * c:3 * c]
    q3 = b[:, 3 * c:4 * c]
    if mode == "px":
        e = _shift(jnp.concatenate([q0, q2], axis=1), d - 1, mode)
        o0, o2 = q1, q3
        o1, o3 = e[:, :c], e[:, c:]
    elif mode == "mx":
        e = _shift(jnp.concatenate([q1, q3], axis=1), d - 1, mode)
        o1, o3 = q0, q2
        o0, o2 = e[:, :c], e[:, c:]
    elif mode == "py":
        e = _shift(jnp.concatenate([q0, q1], axis=1), d - 1, mode)
        o0, o1 = q2, q3
        o2, o3 = e[:, :c], e[:, c:]
    else:
        e = _shift(jnp.concatenate([q2, q3], axis=1), d - 1, mode)
        o2, o3 = q0, q1
        o0, o1 = e[:, :c], e[:, c:]
    return jnp.concatenate([o0, o1, o2, o3], axis=1).reshape(4 ** d, c)


def _mm(a, b):
    return jax.lax.dot_general(a, b, (((1,), (0,)), ((), ())),
                               preferred_element_type=jnp.float32)


def _conv3x3(h, d, cw_ref, cb_ref):
    xm = _shift(h, d, "mx")
    xp = _shift(h, d, "px")
    rows = []
    for r in range(3):
        t = (_mm(xm, cw_ref[d - 1, (3 * r + 0) * _H:(3 * r + 1) * _H, :])
             + _mm(h, cw_ref[d - 1, (3 * r + 1) * _H:(3 * r + 2) * _H, :])
             + _mm(xp, cw_ref[d - 1, (3 * r + 2) * _H:(3 * r + 3) * _H, :]))
        rows.append(t)
    acc = (rows[1] + _shift(rows[0], d, "my") + _shift(rows[2], d, "py")
           + cb_ref[d].reshape(1, _H))
    return jnp.maximum(acc, 0.0)


def _emb_ln(h, d, ew_ref, eb_ref, lg_ref, lb_ref, dg_ref):
    z = _mm(h, ew_ref[d]) + eb_ref[d].reshape(1, _H)
    mu = jnp.mean(z, axis=-1, keepdims=True)
    zc = z - mu
    var = jnp.mean(zc * zc, axis=-1, keepdims=True)
    zn = zc * jax.lax.rsqrt(var + 1e-5)
    return (zn * lg_ref[d].reshape(1, _H) + lb_ref[d].reshape(1, _H)) * dg_ref[d]


def _body(x0, x1, x2, x3, x4, x5, x6, x7, wp_ref, bp_ref, cw_hbm, cb_ref,
          ew_ref, eb_ref, lg_ref, lb_ref, dg_ref,
          o0, o1, o2, o3, o4, o5, o6, o7,
          cw_ref, o6_v, o7_v, sem_w, sem6, sem7):
    xs = (x0, x1, x2, x3, x4, x5, x6, x7)
    outs = (o0, o1, o2, o3, o4, o5, o6_v, o7_v)
    cw_cp = pltpu.make_async_copy(cw_hbm, cw_ref, sem_w)
    cw_cp.start()
    wp = wp_ref[:]
    bp = bp_ref[:]
    pooled = None
    for d in range(_MAXD, -1, -1):
        h = _mm(xs[d][:], wp) + bp
        if pooled is not None:
            h = h + pooled
        if 1 <= d <= 6:
            if d == 6:
                cw_cp.wait()
            h = _conv3x3(h, d, cw_ref, cb_ref)
        outs[d][:] = _emb_ln(h, d, ew_ref, eb_ref, lg_ref, lb_ref, dg_ref)
        if d == 7:
            o7_cp = pltpu.make_async_copy(o7_v, o7, sem7)
            o7_cp.start()
        elif d == 6:
            o6_cp = pltpu.make_async_copy(o6_v, o6, sem6)
            o6_cp.start()
        if d > 0:
            hp = h.reshape(4 ** (d - 1), 4 * _H)
            pooled = (hp[:, 0 * _H:1 * _H] + hp[:, 1 * _H:2 * _H]
                      + hp[:, 2 * _H:3 * _H] + hp[:, 3 * _H:4 * _H]) * 0.25
    o7_cp.wait()
    o6_cp.wait()


def kernel(features_0, features_1, features_2, features_3, features_4,
           features_5, features_6, features_7, in_proj_W, in_proj_b,
           conv_W, conv_b, emb_W, emb_b, ln_g, ln_b, depth_gain):
    feats = (features_0, features_1, features_2, features_3, features_4,
             features_5, features_6, features_7)
    xs = [jnp.concatenate([f, jnp.asarray(_POS[d])], axis=1)
          for d, f in enumerate(feats)]
    vmem = pl.BlockSpec(memory_space=pltpu.MemorySpace.VMEM)
    smem = pl.BlockSpec(memory_space=pltpu.MemorySpace.SMEM)
    hbm = pl.BlockSpec(memory_space=pltpu.MemorySpace.HBM)
    outs = pl.pallas_call(
        _body,
        out_shape=[jax.ShapeDtypeStruct((4 ** d, _H), jnp.float32)
                   for d in range(_MAXD + 1)],
        in_specs=[vmem] * 10 + [hbm] + [vmem] * 5 + [smem],
        out_specs=[vmem] * 6 + [hbm, hbm],
        scratch_shapes=[
            pltpu.VMEM((6, 9 * _H, _H), jnp.float32),
            pltpu.VMEM((4 ** 6, _H), jnp.float32),
            pltpu.VMEM((4 ** 7, _H), jnp.float32),
            pltpu.SemaphoreType.DMA,
            pltpu.SemaphoreType.DMA,
            pltpu.SemaphoreType.DMA,
        ],
    )(*xs, in_proj_W, in_proj_b.reshape(1, _H), conv_W[1:7], conv_b,
      emb_W, emb_b, ln_g, ln_b, depth_gain)
    return tuple(outs)


# speedup vs baseline: 8.9943x; 1.4282x over previous
import numpy as np
import jax
import jax.numpy as jnp
from jax.experimental import pallas as pl
from jax.experimental.pallas import tpu as pltpu

_MAXD = 7
_H = 128
_PF = 6


def _deint(x):
    x = x & 0x55555555
    x = (x | (x >> 1)) & 0x33333333
    x = (x | (x >> 2)) & 0x0F0F0F0F
    x = (x | (x >> 4)) & 0x00FF00FF
    x = (x | (x >> 8)) & 0x0000FFFF
    return x


def _pos_const(d):
    keys = np.arange(4 ** d, dtype=np.int64)
    ix = _deint(keys).astype(np.float32)
    iy = _deint(keys >> 1).astype(np.float32)
    res = np.float32(1 << d)
    x = (ix + np.float32(0.5)) / res
    y = (iy + np.float32(0.5)) / res
    dn = np.full_like(x, np.float32(d) / np.float32(_MAXD))
    pos = np.stack([x, y, dn], axis=1).astype(np.float32)
    freqs = (np.float32(2.0) ** np.arange(_PF, dtype=np.float32)).reshape(1, 1, -1)
    xx = pos[..., None] * np.pi * 2.0 * freqs
    enc = np.concatenate([np.sin(xx), np.cos(xx)], axis=-1).astype(np.float32)
    enc = enc.reshape(pos.shape[0], -1)
    return np.concatenate([pos, enc], axis=1).astype(np.float32)


_POS_T = [np.ascontiguousarray(_pos_const(d).T) for d in range(_MAXD + 1)]
_OFF = [(4 ** d - 1) // 3 for d in range(_MAXD + 2)]


def _shift(a, d, mode):
    c = a.shape[1]
    if d == 0:
        return jnp.zeros_like(a)
    b = a.reshape(4 ** (d - 1), 4 * c)
    q0 = b[:, 0 * c:1 * c]
    q1 = b[:, 1 * c:2 * c]
    q2 = b[:, 2 * c:3 * c]
    q3 = b[:, 3 * c:4 * c]
    if mode == "px":
        e = _shift(jnp.concatenate([q0, q2], axis=1), d - 1, mode)
        o0, o2 = q1, q3
        o1, o3 = e[:, :c], e[:, c:]
    elif mode == "mx":
        e = _shift(jnp.concatenate([q1, q3], axis=1), d - 1, mode)
        o1, o3 = q0, q2
        o0, o2 = e[:, :c], e[:, c:]
    elif mode == "py":
        e = _shift(jnp.concatenate([q0, q1], axis=1), d - 1, mode)
        o0, o1 = q2, q3
        o2, o3 = e[:, :c], e[:, c:]
    else:
        e = _shift(jnp.concatenate([q2, q3], axis=1), d - 1, mode)
        o2, o3 = q0, q1
        o0, o1 = e[:, :c], e[:, c:]
    return jnp.concatenate([o0, o1, o2, o3], axis=1).reshape(4 ** d, c)


def _mm(a, b):
    return jax.lax.dot_general(a, b, (((1,), (0,)), ((), ())),
                               preferred_element_type=jnp.float32)


def _mmT(a, b):
    return jax.lax.dot_general(a, b, (((0,), (0,)), ((), ())),
                               preferred_element_type=jnp.float32)


def _conv3x3(h, d, cw_ref, cb_ref):
    xm = _shift(h, d, "mx")
    xp = _shift(h, d, "px")
    rows = []
    for r in range(3):
        t = (_mm(xm, cw_ref[d - 1, (3 * r + 0) * _H:(3 * r + 1) * _H, :])
             + _mm(h, cw_ref[d - 1, (3 * r + 1) * _H:(3 * r + 2) * _H, :])
             + _mm(xp, cw_ref[d - 1, (3 * r + 2) * _H:(3 * r + 3) * _H, :]))
        rows.append(t)
    acc = (rows[1] + _shift(rows[0], d, "my") + _shift(rows[2], d, "py")
           + cb_ref[d].reshape(1, _H))
    return jnp.maximum(acc, 0.0)


def _emb_ln(h, d, ew_ref, eb_ref, lg_ref, lb_ref, dg_ref):
    z = _mm(h, ew_ref[d]) + eb_ref[d].reshape(1, _H)
    mu = jnp.mean(z, axis=-1, keepdims=True)
    zc = z - mu
    var = jnp.mean(zc * zc, axis=-1, keepdims=True)
    zn = zc * jax.lax.rsqrt(var + 1e-5)
    return (zn * lg_ref[d].reshape(1, _H) + lb_ref[d].reshape(1, _H)) * dg_ref[d]


def _body(fall_ref, p0, p1, p2, p3, p4, p5, p6, p7, wp_ref, bp_ref, cw_hbm,
          cb_ref, ew_ref, eb_ref, lg_ref, lb_ref, dg_ref,
          o0, o1, o2, o3, o4, o5, o6, o7,
          cw_ref, o6_v, o7_v, sem_w, sem6, sem7):
    ps = (p0, p1, p2, p3, p4, p5, p6, p7)
    outs = (o0, o1, o2, o3, o4, o5, o6_v, o7_v)
    cw_cp = pltpu.make_async_copy(cw_hbm, cw_ref, sem_w)
    cw_cp.start()
    wp = wp_ref[:]
    wpf = wp[0:1, :]
    wpp = wp[1:, :]
    bp = bp_ref[:]
    pooled = None
    for d in range(_MAXD, -1, -1):
        n = 4 ** d
        fT = fall_ref[:, _OFF[d]:_OFF[d + 1]]
        h = _mmT(ps[d][:], wpp) + _mmT(fT, wpf) + bp
        if pooled is not None:
            h = h + pooled
        if 1 <= d <= 6:
            if d == 6:
                cw_cp.wait()
            h = _conv3x3(h, d, cw_ref, cb_ref)
        outs[d][:] = _emb_ln(h, d, ew_ref, eb_ref, lg_ref, lb_ref, dg_ref)
        if d == 7:
            o7_cp = pltpu.make_async_copy(o7_v, o7, sem7)
            o7_cp.start()
        elif d == 6:
            o6_cp = pltpu.make_async_copy(o6_v, o6, sem6)
            o6_cp.start()
        if d > 0:
            hp = h.reshape(n // 4, 4 * _H)
            pooled = (hp[:, 0 * _H:1 * _H] + hp[:, 1 * _H:2 * _H]
                      + hp[:, 2 * _H:3 * _H] + hp[:, 3 * _H:4 * _H]) * 0.25
    o7_cp.wait()
    o6_cp.wait()


def kernel(features_0, features_1, features_2, features_3, features_4,
           features_5, features_6, features_7, in_proj_W, in_proj_b,
           conv_W, conv_b, emb_W, emb_b, ln_g, ln_b, depth_gain):
    feats = (features_0, features_1, features_2, features_3, features_4,
             features_5, features_6, features_7)
    fall = jnp.concatenate([f.reshape(1, -1) for f in feats], axis=1)
    pos = [jnp.asarray(p) for p in _POS_T]
    vmem = pl.BlockSpec(memory_space=pltpu.MemorySpace.VMEM)
    smem = pl.BlockSpec(memory_space=pltpu.MemorySpace.SMEM)
    hbm = pl.BlockSpec(memory_space=pltpu.MemorySpace.HBM)
    outs = pl.pallas_call(
        _body,
        out_shape=[jax.ShapeDtypeStruct((4 ** d, _H), jnp.float32)
                   for d in range(_MAXD + 1)],
        in_specs=[vmem] * 11 + [hbm] + [vmem] * 5 + [smem],
        out_specs=[vmem] * 6 + [hbm, hbm],
        scratch_shapes=[
            pltpu.VMEM((6, 9 * _H, _H), jnp.float32),
            pltpu.VMEM((4 ** 6, _H), jnp.float32),
            pltpu.VMEM((4 ** 7, _H), jnp.float32),
            pltpu.SemaphoreType.DMA,
            pltpu.SemaphoreType.DMA,
            pltpu.SemaphoreType.DMA,
        ],
    )(fall, *pos, in_proj_W, in_proj_b.reshape(1, _H), conv_W[1:7], conv_b,
      emb_W, emb_b, ln_g, ln_b, depth_gain)
    return tuple(outs)
